# Initial kernel scaffold; baseline (speedup 1.0000x reference)
import jax
import jax.numpy as jnp
from jax.experimental import pallas as pl


def kernel(coords, features, params, batch_ids):
    raise NotImplementedError("write your pallas kernel here")


# speedup vs baseline: 7.4987x; 7.4987x over previous
import functools

import jax
import jax.numpy as jnp
from jax.experimental import pallas as pl
from jax.experimental.pallas import tpu as pltpu

B = 16
N = 32768
D_IN = 131
D_H = 256
NUM_EXP = 6
D_HALL = D_H * NUM_EXP
D_OUT = 6 + 2 + 2 + 3 + 3 + 3
BLK = 2048


def _fused_kernel(x_ref, ids_ref, w1_ref, b1_ref, w2_ref, b2_ref, out_ref,
                  acc_ref, cnt_ref):
    i = pl.program_id(0)
    nsteps = pl.num_programs(0)

    @pl.when(i == 0)
    def _init():
        acc_ref[...] = jnp.zeros_like(acc_ref)
        cnt_ref[...] = jnp.zeros_like(cnt_ref)

    x = x_ref[...]
    h = jnp.dot(x, w1_ref[...], preferred_element_type=jnp.float32)
    h = jax.nn.gelu(h + b1_ref[...])

    ids = ids_ref[0]
    seg = jax.lax.broadcasted_iota(jnp.int32, (B, BLK), 0)
    oh_t = (seg == ids).astype(jnp.float32)

    acc_ref[...] += jnp.dot(oh_t, h, preferred_element_type=jnp.float32)
    cnt_ref[...] += jnp.sum(oh_t, axis=1, keepdims=True)

    @pl.when(i == nsteps - 1)
    def _finish():
        pooled = acc_ref[...] / jnp.maximum(cnt_ref[...], 1.0)
        raw = jnp.dot(pooled, w2_ref[...],
                      preferred_element_type=jnp.float32) + b2_ref[...]
        morph = raw[:, 0:6]
        m = jnp.max(morph, axis=-1, keepdims=True)
        e = jnp.exp(morph - m)
        p = e / jnp.sum(e, axis=-1, keepdims=True)
        p = jnp.clip(p, 1e-06, None)
        p_cont = p[:, 0:1] + p[:, 1:2]
        p_uncont = p[:, 2:3] + p[:, 3:4] + p[:, 5:6]
        energy = p_cont * raw[:, 6:8] + p_uncont * raw[:, 8:10]
        gate = jax.nn.sigmoid(energy[:, 0:1] - 4.0)
        p_cas = p[:, 0:1]
        p_track = p[:, 1:2] + p[:, 2:3] + p[:, 3:4] + p[:, 5:6]
        dir_pred = (p_cas * raw[:, 10:13]
                    + p_track * (1.0 - gate) * raw[:, 13:16]
                    + p_track * gate * raw[:, 16:19])
        out_ref[...] = jnp.concatenate([morph, energy, dir_pred], axis=1)


@functools.partial(jax.jit, static_argnames=())
def _run(x, ids3, w1_all, b1_all, w2_big, b2_all):
    nblk = N // BLK
    return pl.pallas_call(
        _fused_kernel,
        grid=(nblk,),
        in_specs=[
            pl.BlockSpec((BLK, D_IN), lambda i: (i, 0)),
            pl.BlockSpec((1, 1, BLK), lambda i: (i, 0, 0)),
            pl.BlockSpec((D_IN, D_HALL), lambda i: (0, 0)),
            pl.BlockSpec((1, D_HALL), lambda i: (0, 0)),
            pl.BlockSpec((D_HALL, D_OUT), lambda i: (0, 0)),
            pl.BlockSpec((1, D_OUT), lambda i: (0, 0)),
        ],
        out_specs=pl.BlockSpec((B, 11), lambda i: (0, 0)),
        out_shape=jax.ShapeDtypeStruct((B, 11), jnp.float32),
        scratch_shapes=[
            pltpu.VMEM((B, D_HALL), jnp.float32),
            pltpu.VMEM((B, 1), jnp.float32),
        ],
    )(x, ids3, w1_all, b1_all, w2_big, b2_all)


def kernel(coords, features, params, batch_ids):
    x = jnp.concatenate([coords, features], axis=-1)
    order = ("router", "e_contained", "e_uncontained",
             "d_cascade", "d_low", "d_high")
    w1_all = jnp.concatenate([params[k]["W1"] for k in order], axis=1)
    b1_all = jnp.concatenate([params[k]["b1"] for k in order])[None, :]
    outs = [6, 2, 2, 3, 3, 3]
    w2_big = jnp.zeros((D_HALL, D_OUT), jnp.float32)
    col = 0
    for k, name in enumerate(order):
        w2 = params[name]["W2"]
        w2_big = w2_big.at[k * D_H:(k + 1) * D_H, col:col + outs[k]].set(w2)
        col += outs[k]
    b2_all = jnp.concatenate([params[k]["b2"] for k in order])[None, :]
    ids3 = batch_ids.astype(jnp.int32).reshape(N // BLK, 1, BLK)
    return _run(x, ids3, w1_all, b1_all, w2_big, b2_all)


# speedup vs baseline: 10.3175x; 1.3759x over previous
import functools

import jax
import jax.numpy as jnp
from jax.experimental import pallas as pl
from jax.experimental.pallas import tpu as pltpu

B = 16
N = 32768
D_IN = 131
D_H = 256
NUM_EXP = 6
D_HALL = D_H * NUM_EXP
D_OUT = 6 + 2 + 2 + 3 + 3 + 3
BLK = 4096

_GELU_A = 0.7978845608028654
_GELU_C = 0.7978845608028654 * 0.044715


def _gelu(x):
    a = jnp.asarray(_GELU_A, x.dtype)
    c = jnp.asarray(_GELU_C, x.dtype)
    half = jnp.asarray(0.5, x.dtype)
    one = jnp.asarray(1.0, x.dtype)
    u = x * (a + c * (x * x))
    return half * x * (one + jnp.tanh(u))


def _fused_kernel(x_ref, ids_ref, w1_ref, b1_ref, w2_ref, b2_ref,
                  out_ref, acc_ref, cnt_ref):
    i = pl.program_id(0)
    nsteps = pl.num_programs(0)

    @pl.when(i == 0)
    def _init():
        acc_ref[...] = jnp.zeros_like(acc_ref)
        cnt_ref[...] = jnp.zeros_like(cnt_ref)

    h = jnp.dot(x_ref[...], w1_ref[...],
                preferred_element_type=jnp.float32).astype(jnp.bfloat16)
    h = _gelu(h + b1_ref[...])

    ids = ids_ref[0]
    seg = jax.lax.broadcasted_iota(jnp.int32, (B, BLK), 0)
    oh_t = (seg == ids).astype(jnp.bfloat16)

    acc_ref[...] += jnp.dot(oh_t, h, preferred_element_type=jnp.float32)
    cnt_ref[...] += jnp.sum(oh_t.astype(jnp.float32), axis=1, keepdims=True)

    @pl.when(i == nsteps - 1)
    def _finish():
        pooled = acc_ref[...] / jnp.maximum(cnt_ref[...], 1.0)
        raw = jnp.dot(pooled, w2_ref[...],
                      preferred_element_type=jnp.float32) + b2_ref[...]
        morph = raw[:, 0:6]
        m = jnp.max(morph, axis=-1, keepdims=True)
        e = jnp.exp(morph - m)
        p = e / jnp.sum(e, axis=-1, keepdims=True)
        p = jnp.clip(p, 1e-06, None)
        p_cont = p[:, 0:1] + p[:, 1:2]
        p_uncont = p[:, 2:3] + p[:, 3:4] + p[:, 5:6]
        energy = p_cont * raw[:, 6:8] + p_uncont * raw[:, 8:10]
        gate = jax.nn.sigmoid(energy[:, 0:1] - 4.0)
        p_cas = p[:, 0:1]
        p_track = p[:, 1:2] + p[:, 2:3] + p[:, 3:4] + p[:, 5:6]
        dir_pred = (p_cas * raw[:, 10:13]
                    + p_track * (1.0 - gate) * raw[:, 13:16]
                    + p_track * gate * raw[:, 16:19])
        out_ref[...] = jnp.concatenate([morph, energy, dir_pred], axis=1)


@jax.jit
def _run(x, ids3, w1_all, b1_all, w2_big, b2_all):
    nblk = N // BLK
    return pl.pallas_call(
        _fused_kernel,
        grid=(nblk,),
        in_specs=[
            pl.BlockSpec((BLK, D_IN), lambda i: (i, 0)),
            pl.BlockSpec((1, 1, BLK), lambda i: (i, 0, 0)),
            pl.BlockSpec((D_IN, D_HALL), lambda i: (0, 0)),
            pl.BlockSpec((1, D_HALL), lambda i: (0, 0)),
            pl.BlockSpec((D_HALL, D_OUT), lambda i: (0, 0)),
            pl.BlockSpec((1, D_OUT), lambda i: (0, 0)),
        ],
        out_specs=pl.BlockSpec((B, 11), lambda i: (0, 0)),
        out_shape=jax.ShapeDtypeStruct((B, 11), jnp.float32),
        scratch_shapes=[
            pltpu.VMEM((B, D_HALL), jnp.float32),
            pltpu.VMEM((B, 1), jnp.float32),
        ],
    )(x, ids3, w1_all, b1_all, w2_big, b2_all)


def kernel(coords, features, params, batch_ids):
    order = ("router", "e_contained", "e_uncontained",
             "d_cascade", "d_low", "d_high")
    x = jnp.concatenate([coords, features], axis=-1).astype(jnp.bfloat16)
    w1_all = jnp.concatenate(
        [params[k]["W1"] for k in order], axis=1).astype(jnp.bfloat16)
    b1_all = jnp.concatenate(
        [params[k]["b1"] for k in order])[None, :].astype(jnp.bfloat16)
    outs = [6, 2, 2, 3, 3, 3]
    w2_big = jnp.zeros((D_HALL, D_OUT), jnp.float32)
    col = 0
    for k, name in enumerate(order):
        w2 = params[name]["W2"]
        w2_big = w2_big.at[k * D_H:(k + 1) * D_H, col:col + outs[k]].set(w2)
        col += outs[k]
    b2_all = jnp.concatenate([params[k]["b2"] for k in order])[None, :]
    ids3 = batch_ids.astype(jnp.int32).reshape(N // BLK, 1, BLK)
    return _run(x, ids3, w1_all, b1_all, w2_big, b2_all)


# speedup vs baseline: 10.3773x; 1.0058x over previous
import functools

import jax
import jax.numpy as jnp
from jax.experimental import pallas as pl
from jax.experimental.pallas import tpu as pltpu

B = 16
N = 32768
D_IN = 132
D_H = 256
NUM_EXP = 6
D_HALL = D_H * NUM_EXP
D_OUT = 6 + 2 + 2 + 3 + 3 + 3
BLK = 8192

_GELU_A = 0.7978845608028654
_GELU_C = 0.7978845608028654 * 0.044715


def _gelu(x):
    a = jnp.asarray(_GELU_A, x.dtype)
    c = jnp.asarray(_GELU_C, x.dtype)
    half = jnp.asarray(0.5, x.dtype)
    one = jnp.asarray(1.0, x.dtype)
    u = x * (a + c * (x * x))
    return half * x * (one + jnp.tanh(u))


def _fused_kernel(x_ref, ids_ref, w1_ref, w2_ref, b2_ref,
                  out_ref, acc_ref, cnt_ref):
    i = pl.program_id(0)
    nsteps = pl.num_programs(0)

    @pl.when(i == 0)
    def _init():
        acc_ref[...] = jnp.zeros_like(acc_ref)
        cnt_ref[...] = jnp.zeros_like(cnt_ref)

    ids = ids_ref[0]
    seg = jax.lax.broadcasted_iota(jnp.int32, (B, BLK), 0)
    oh_t = (seg == ids).astype(jnp.bfloat16)
    x = x_ref[...]

    HC = D_HALL // 2
    h1 = jnp.dot(x, w1_ref[:, :HC],
                 preferred_element_type=jnp.float32).astype(jnp.bfloat16)
    g1 = _gelu(h1)
    h2 = jnp.dot(x, w1_ref[:, HC:],
                 preferred_element_type=jnp.float32).astype(jnp.bfloat16)
    g2 = _gelu(h2)
    acc_ref[:, :HC] += jnp.dot(oh_t, g1, preferred_element_type=jnp.float32)
    acc_ref[:, HC:] += jnp.dot(oh_t, g2, preferred_element_type=jnp.float32)
    cnt_ref[...] += jnp.sum(oh_t.astype(jnp.float32), axis=1, keepdims=True)

    @pl.when(i == nsteps - 1)
    def _finish():
        pooled = acc_ref[...] / jnp.maximum(cnt_ref[...], 1.0)
        raw = jnp.dot(pooled, w2_ref[...],
                      preferred_element_type=jnp.float32) + b2_ref[...]
        morph = raw[:, 0:6]
        m = jnp.max(morph, axis=-1, keepdims=True)
        e = jnp.exp(morph - m)
        p = e / jnp.sum(e, axis=-1, keepdims=True)
        p = jnp.clip(p, 1e-06, None)
        p_cont = p[:, 0:1] + p[:, 1:2]
        p_uncont = p[:, 2:3] + p[:, 3:4] + p[:, 5:6]
        energy = p_cont * raw[:, 6:8] + p_uncont * raw[:, 8:10]
        gate = jax.nn.sigmoid(energy[:, 0:1] - 4.0)
        p_cas = p[:, 0:1]
        p_track = p[:, 1:2] + p[:, 2:3] + p[:, 3:4] + p[:, 5:6]
        dir_pred = (p_cas * raw[:, 10:13]
                    + p_track * (1.0 - gate) * raw[:, 13:16]
                    + p_track * gate * raw[:, 16:19])
        out_ref[...] = jnp.concatenate([morph, energy, dir_pred], axis=1)


@jax.jit
def _run(x, ids3, w1_all, w2_big, b2_all):
    nblk = N // BLK
    return pl.pallas_call(
        _fused_kernel,
        grid=(nblk,),
        in_specs=[
            pl.BlockSpec((BLK, D_IN), lambda i: (i, 0)),
            pl.BlockSpec((1, 1, BLK), lambda i: (i, 0, 0)),
            pl.BlockSpec((D_IN, D_HALL), lambda i: (0, 0)),
            pl.BlockSpec((D_HALL, D_OUT), lambda i: (0, 0)),
            pl.BlockSpec((1, D_OUT), lambda i: (0, 0)),
        ],
        out_specs=pl.BlockSpec((B, 11), lambda i: (0, 0)),
        out_shape=jax.ShapeDtypeStruct((B, 11), jnp.float32),
        scratch_shapes=[
            pltpu.VMEM((B, D_HALL), jnp.float32),
            pltpu.VMEM((B, 1), jnp.float32),
        ],
    )(x, ids3, w1_all, w2_big, b2_all)


def kernel(coords, features, params, batch_ids):
    order = ("router", "e_contained", "e_uncontained",
             "d_cascade", "d_low", "d_high")
    x = jnp.concatenate(
        [coords, features, jnp.ones((N, 1), jnp.float32)],
        axis=-1).astype(jnp.bfloat16)
    w1_all = jnp.concatenate([params[k]["W1"] for k in order], axis=1)
    b1_all = jnp.concatenate([params[k]["b1"] for k in order])[None, :]
    w1_all = jnp.concatenate([w1_all, b1_all], axis=0).astype(jnp.bfloat16)
    outs = [6, 2, 2, 3, 3, 3]
    w2_big = jnp.zeros((D_HALL, D_OUT), jnp.float32)
    col = 0
    for k, name in enumerate(order):
        w2 = params[name]["W2"]
        w2_big = w2_big.at[k * D_H:(k + 1) * D_H, col:col + outs[k]].set(w2)
        col += outs[k]
    b2_all = jnp.concatenate([params[k]["b2"] for k in order])[None, :]
    ids3 = batch_ids.astype(jnp.int32).reshape(N // BLK, 1, BLK)
    return _run(x, ids3, w1_all, w2_big, b2_all)


# speedup vs baseline: 11.7454x; 1.1318x over previous
import jax
import jax.numpy as jnp
from jax.experimental import pallas as pl
from jax.experimental.pallas import tpu as pltpu

B = 16
N = 32768
D_IN = 132
D_H = 256
NUM_EXP = 6
D_HALL = D_H * NUM_EXP
BLK = 8192

_GELU_A = 0.7978845608028654
_GELU_C = 0.7978845608028654 * 0.044715


def _gelu(x):
    a = jnp.asarray(_GELU_A, x.dtype)
    c = jnp.asarray(_GELU_C, x.dtype)
    half = jnp.asarray(0.5, x.dtype)
    one = jnp.asarray(1.0, x.dtype)
    u = x * (a + c * (x * x))
    return half * x * (one + jnp.tanh(u))


def _fused_kernel(x_ref, ids_ref,
                  w1r_ref, w1c_ref, w1u_ref, w1dc_ref, w1dl_ref, w1dh_ref,
                  b1r_ref, b1c_ref, b1u_ref, b1dc_ref, b1dl_ref, b1dh_ref,
                  w2r_ref, w2c_ref, w2u_ref, w2dc_ref, w2dl_ref, w2dh_ref,
                  b2r_ref, b2c_ref, b2u_ref, b2dc_ref, b2dl_ref, b2dh_ref,
                  out_ref, w1s_ref, acc_ref, cnt_ref):
    i = pl.program_id(0)
    nsteps = pl.num_programs(0)

    @pl.when(i == 0)
    def _init():
        acc_ref[...] = jnp.zeros_like(acc_ref)
        cnt_ref[...] = jnp.zeros_like(cnt_ref)
        w1_refs = (w1r_ref, w1c_ref, w1u_ref, w1dc_ref, w1dl_ref, w1dh_ref)
        b1_refs = (b1r_ref, b1c_ref, b1u_ref, b1dc_ref, b1dl_ref, b1dh_ref)
        for k in range(NUM_EXP):
            c0 = k * D_H
            w1s_ref[0:D_IN - 1, c0:c0 + D_H] = (
                w1_refs[k][...].astype(jnp.bfloat16))
            w1s_ref[D_IN - 1:D_IN, c0:c0 + D_H] = (
                b1_refs[k][...].astype(jnp.bfloat16))

    ids = ids_ref[0]
    seg = jax.lax.broadcasted_iota(jnp.int32, (B, BLK), 0)
    oh_t = (seg == ids).astype(jnp.bfloat16)
    x = x_ref[...]

    HC = D_HALL // 2
    h1 = jnp.dot(x, w1s_ref[:, :HC],
                 preferred_element_type=jnp.float32).astype(jnp.bfloat16)
    g1 = _gelu(h1)
    h2 = jnp.dot(x, w1s_ref[:, HC:],
                 preferred_element_type=jnp.float32).astype(jnp.bfloat16)
    g2 = _gelu(h2)
    acc_ref[:, :HC] += jnp.dot(oh_t, g1, preferred_element_type=jnp.float32)
    acc_ref[:, HC:] += jnp.dot(oh_t, g2, preferred_element_type=jnp.float32)
    cnt_ref[...] += jnp.sum(oh_t.astype(jnp.float32), axis=1, keepdims=True)

    @pl.when(i == nsteps - 1)
    def _finish():
        pooled = acc_ref[...] / jnp.maximum(cnt_ref[...], 1.0)

        w2_refs = (w2r_ref, w2c_ref, w2u_ref, w2dc_ref, w2dl_ref, w2dh_ref)
        b2_refs = (b2r_ref, b2c_ref, b2u_ref, b2dc_ref, b2dl_ref, b2dh_ref)
        raw = []
        for k in range(NUM_EXP):
            pk = pooled[:, k * D_H:(k + 1) * D_H]
            raw.append(jnp.dot(pk, w2_refs[k][...],
                               preferred_element_type=jnp.float32)
                       + b2_refs[k][...])
        morph, e_cont, e_uncont, d_cas, d_low, d_high = raw

        m = jnp.max(morph, axis=-1, keepdims=True)
        e = jnp.exp(morph - m)
        p = e / jnp.sum(e, axis=-1, keepdims=True)
        p = jnp.clip(p, 1e-06, None)
        p_cont = p[:, 0:1] + p[:, 1:2]
        p_uncont = p[:, 2:3] + p[:, 3:4] + p[:, 5:6]
        energy = p_cont * e_cont + p_uncont * e_uncont
        gate = jax.nn.sigmoid(energy[:, 0:1] - 4.0)
        p_cas = p[:, 0:1]
        p_track = p[:, 1:2] + p[:, 2:3] + p[:, 3:4] + p[:, 5:6]
        dir_pred = (p_cas * d_cas
                    + p_track * (1.0 - gate) * d_low
                    + p_track * gate * d_high)
        out_ref[...] = jnp.concatenate([morph, energy, dir_pred], axis=1)


def _full(shape):
    nd = len(shape)
    return pl.BlockSpec(shape, lambda i: (0,) * nd)


@jax.jit
def _run(x, ids3, w1s, b1s, w2s, b2s):
    nblk = N // BLK
    in_specs = (
        [pl.BlockSpec((BLK, D_IN), lambda i: (i, 0)),
         pl.BlockSpec((1, 1, BLK), lambda i: (i, 0, 0))]
        + [_full(w.shape) for w in w1s]
        + [_full(b.shape) for b in b1s]
        + [_full(w.shape) for w in w2s]
        + [_full(b.shape) for b in b2s]
    )
    return pl.pallas_call(
        _fused_kernel,
        grid=(nblk,),
        in_specs=in_specs,
        out_specs=pl.BlockSpec((B, 11), lambda i: (0, 0)),
        out_shape=jax.ShapeDtypeStruct((B, 11), jnp.float32),
        scratch_shapes=[
            pltpu.VMEM((D_IN, D_HALL), jnp.bfloat16),
            pltpu.VMEM((B, D_HALL), jnp.float32),
            pltpu.VMEM((B, 1), jnp.float32),
        ],
    )(x, ids3, *w1s, *b1s, *w2s, *b2s)


def kernel(coords, features, params, batch_ids):
    order = ("router", "e_contained", "e_uncontained",
             "d_cascade", "d_low", "d_high")
    x = jnp.concatenate(
        [coords, features, jnp.ones((N, 1), jnp.float32)],
        axis=-1).astype(jnp.bfloat16)
    ids3 = batch_ids.astype(jnp.int32).reshape(N // BLK, 1, BLK)
    w1s = [params[k]["W1"] for k in order]
    b1s = [params[k]["b1"][None, :] for k in order]
    w2s = [params[k]["W2"] for k in order]
    b2s = [params[k]["b2"][None, :] for k in order]
    return _run(x, ids3, w1s, b1s, w2s, b2s)


# speedup vs baseline: 12.5205x; 1.0660x over previous
import jax
import jax.numpy as jnp
from jax.experimental import pallas as pl
from jax.experimental.pallas import tpu as pltpu

B = 16
N = 32768
D_IN = 132
D_H = 256
NUM_EXP = 6
D_HALL = D_H * NUM_EXP
BLK = 8192
NCHUNK = 2

_GELU_A = 0.7978845608028654
_GELU_C = 0.7978845608028654 * 0.044715


def _gelu2(x):
    a = jnp.asarray(_GELU_A, x.dtype)
    c = jnp.asarray(_GELU_C, x.dtype)
    one = jnp.asarray(1.0, x.dtype)
    u = x * (a + c * (x * x))
    return x * (one + jnp.tanh(u))


def _fused_kernel(x_ref, ids_ref,
                  w1r_ref, w1c_ref, w1u_ref, w1dc_ref, w1dl_ref, w1dh_ref,
                  b1r_ref, b1c_ref, b1u_ref, b1dc_ref, b1dl_ref, b1dh_ref,
                  w2r_ref, w2c_ref, w2u_ref, w2dc_ref, w2dl_ref, w2dh_ref,
                  b2r_ref, b2c_ref, b2u_ref, b2dc_ref, b2dl_ref, b2dh_ref,
                  out_ref, w1s_ref, acc_ref, cnt_ref):
    i = pl.program_id(0)
    nsteps = pl.num_programs(0)

    @pl.when(i == 0)
    def _init():
        acc_ref[...] = jnp.zeros_like(acc_ref)
        cnt_ref[...] = jnp.zeros_like(cnt_ref)
        w1_refs = (w1r_ref, w1c_ref, w1u_ref, w1dc_ref, w1dl_ref, w1dh_ref)
        b1_refs = (b1r_ref, b1c_ref, b1u_ref, b1dc_ref, b1dl_ref, b1dh_ref)
        for k in range(NUM_EXP):
            c0 = k * D_H
            w1s_ref[0:D_IN - 1, c0:c0 + D_H] = (
                w1_refs[k][...].astype(jnp.bfloat16))
            w1s_ref[D_IN - 1:D_IN, c0:c0 + D_H] = (
                b1_refs[k][...].astype(jnp.bfloat16))

    ids = ids_ref[0]
    seg = jax.lax.broadcasted_iota(jnp.int32, (B, BLK), 0)
    oh_t = (seg == ids).astype(jnp.bfloat16)
    x = x_ref[...]

    CW = D_HALL // NCHUNK
    for j in range(NCHUNK):
        hj = jnp.dot(x, w1s_ref[:, j * CW:(j + 1) * CW],
                     preferred_element_type=jnp.float32).astype(jnp.bfloat16)
        gj = _gelu2(hj)
        acc_ref[:, j * CW:(j + 1) * CW] += jnp.dot(
            oh_t, gj, preferred_element_type=jnp.float32)
    cnt_ref[...] += jnp.sum(oh_t.astype(jnp.float32), axis=1, keepdims=True)

    @pl.when(i == nsteps - 1)
    def _finish():
        pooled = acc_ref[...] / (2.0 * jnp.maximum(cnt_ref[...], 1.0))

        w2_refs = (w2r_ref, w2c_ref, w2u_ref, w2dc_ref, w2dl_ref, w2dh_ref)
        b2_refs = (b2r_ref, b2c_ref, b2u_ref, b2dc_ref, b2dl_ref, b2dh_ref)
        raw = []
        for k in range(NUM_EXP):
            pk = pooled[:, k * D_H:(k + 1) * D_H]
            raw.append(jnp.dot(pk, w2_refs[k][...],
                               preferred_element_type=jnp.float32)
                       + b2_refs[k][...])
        morph, e_cont, e_uncont, d_cas, d_low, d_high = raw

        m = jnp.max(morph, axis=-1, keepdims=True)
        e = jnp.exp(morph - m)
        p = e / jnp.sum(e, axis=-1, keepdims=True)
        p = jnp.clip(p, 1e-06, None)
        p_cont = p[:, 0:1] + p[:, 1:2]
        p_uncont = p[:, 2:3] + p[:, 3:4] + p[:, 5:6]
        energy = p_cont * e_cont + p_uncont * e_uncont
        gate = jax.nn.sigmoid(energy[:, 0:1] - 4.0)
        p_cas = p[:, 0:1]
        p_track = p[:, 1:2] + p[:, 2:3] + p[:, 3:4] + p[:, 5:6]
        dir_pred = (p_cas * d_cas
                    + p_track * (1.0 - gate) * d_low
                    + p_track * gate * d_high)
        out_ref[...] = jnp.concatenate([morph, energy, dir_pred], axis=1)


def _full(shape):
    nd = len(shape)
    return pl.BlockSpec(shape, lambda i: (0,) * nd)


@jax.jit
def _run(x, ids3, w1s, b1s, w2s, b2s):
    nblk = N // BLK
    in_specs = (
        [pl.BlockSpec((BLK, D_IN), lambda i: (i, 0)),
         pl.BlockSpec((1, 1, BLK), lambda i: (i, 0, 0))]
        + [_full(w.shape) for w in w1s]
        + [_full(b.shape) for b in b1s]
        + [_full(w.shape) for w in w2s]
        + [_full(b.shape) for b in b2s]
    )
    return pl.pallas_call(
        _fused_kernel,
        grid=(nblk,),
        in_specs=in_specs,
        out_specs=pl.BlockSpec((B, 11), lambda i: (0, 0)),
        out_shape=jax.ShapeDtypeStruct((B, 11), jnp.float32),
        scratch_shapes=[
            pltpu.VMEM((D_IN, D_HALL), jnp.bfloat16),
            pltpu.VMEM((B, D_HALL), jnp.float32),
            pltpu.VMEM((B, 1), jnp.float32),
        ],
    )(x, ids3, *w1s, *b1s, *w2s, *b2s)


def kernel(coords, features, params, batch_ids):
    order = ("router", "e_contained", "e_uncontained",
             "d_cascade", "d_low", "d_high")
    x = jnp.concatenate(
        [coords, features, jnp.ones((N, 1), jnp.float32)],
        axis=-1).astype(jnp.bfloat16)
    ids3 = batch_ids.astype(jnp.int32).reshape(N // BLK, 1, BLK)
    w1s = [params[k]["W1"] for k in order]
    b1s = [params[k]["b1"][None, :] for k in order]
    w2s = [params[k]["W2"] for k in order]
    b2s = [params[k]["b2"][None, :] for k in order]
    return _run(x, ids3, w1s, b1s, w2s, b2s)


# speedup vs baseline: 14.1278x; 1.1284x over previous
import jax
import jax.numpy as jnp
from jax.experimental import pallas as pl
from jax.experimental.pallas import tpu as pltpu

B = 16
N = 32768
D_IN = 132
D_H = 256
NUM_EXP = 6
D_HALL = D_H * NUM_EXP
BLK = 8192
NCHUNK = 2

_GELU_A = 0.7978845608028654
_GELU_C = 0.7978845608028654 * 0.044715


def _gelu2(x):
    a = jnp.asarray(_GELU_A, x.dtype)
    c = jnp.asarray(_GELU_C, x.dtype)
    one = jnp.asarray(1.0, x.dtype)
    u = x * (a + c * (x * x))
    return x * (one + jnp.tanh(u))


def _fused_kernel(crd_ref, ft_ref, ids_ref,
                  w1r_ref, w1c_ref, w1u_ref, w1dc_ref, w1dl_ref, w1dh_ref,
                  b1r_ref, b1c_ref, b1u_ref, b1dc_ref, b1dl_ref, b1dh_ref,
                  w2r_ref, w2c_ref, w2u_ref, w2dc_ref, w2dl_ref, w2dh_ref,
                  b2r_ref, b2c_ref, b2u_ref, b2dc_ref, b2dl_ref, b2dh_ref,
                  out_ref, w1s_ref, xs_ref, acc_ref, cnt_ref):
    i = pl.program_id(0)
    nsteps = pl.num_programs(0)

    @pl.when(i == 0)
    def _init():
        acc_ref[...] = jnp.zeros_like(acc_ref)
        cnt_ref[...] = jnp.zeros_like(cnt_ref)
        w1_refs = (w1r_ref, w1c_ref, w1u_ref, w1dc_ref, w1dl_ref, w1dh_ref)
        b1_refs = (b1r_ref, b1c_ref, b1u_ref, b1dc_ref, b1dl_ref, b1dh_ref)
        for k in range(NUM_EXP):
            c0 = k * D_H
            wk = w1_refs[k][...].astype(jnp.bfloat16)
            w1s_ref[0:128, c0:c0 + D_H] = wk[3:131, :]
            w1s_ref[128:131, c0:c0 + D_H] = wk[0:3, :]
            w1s_ref[131:132, c0:c0 + D_H] = (
                b1_refs[k][...].astype(jnp.bfloat16))

    ids = ids_ref[0]
    seg = jax.lax.broadcasted_iota(jnp.int32, (B, BLK), 0)
    oh_t = (seg == ids).astype(jnp.bfloat16)

    xs_ref[:, 0:128] = ft_ref[...].astype(jnp.bfloat16)
    xs_ref[:, 128:131] = crd_ref[...].astype(jnp.bfloat16)
    xs_ref[:, 131:132] = jnp.ones((BLK, 1), jnp.bfloat16)
    x = xs_ref[...]

    CW = D_HALL // NCHUNK
    for j in range(NCHUNK):
        hj = jnp.dot(x, w1s_ref[:, j * CW:(j + 1) * CW],
                     preferred_element_type=jnp.float32).astype(jnp.bfloat16)
        gj = _gelu2(hj)
        acc_ref[:, j * CW:(j + 1) * CW] += jnp.dot(
            oh_t, gj, preferred_element_type=jnp.float32)
    cnt_ref[...] += jnp.sum(oh_t.astype(jnp.float32), axis=1, keepdims=True)

    @pl.when(i == nsteps - 1)
    def _finish():
        pooled = acc_ref[...] / (2.0 * jnp.maximum(cnt_ref[...], 1.0))

        w2_refs = (w2r_ref, w2c_ref, w2u_ref, w2dc_ref, w2dl_ref, w2dh_ref)
        b2_refs = (b2r_ref, b2c_ref, b2u_ref, b2dc_ref, b2dl_ref, b2dh_ref)
        raw = []
        for k in range(NUM_EXP):
            pk = pooled[:, k * D_H:(k + 1) * D_H]
            raw.append(jnp.dot(pk, w2_refs[k][...],
                               preferred_element_type=jnp.float32)
                       + b2_refs[k][...])
        morph, e_cont, e_uncont, d_cas, d_low, d_high = raw

        m = jnp.max(morph, axis=-1, keepdims=True)
        e = jnp.exp(morph - m)
        p = e / jnp.sum(e, axis=-1, keepdims=True)
        p = jnp.clip(p, 1e-06, None)
        p_cont = p[:, 0:1] + p[:, 1:2]
        p_uncont = p[:, 2:3] + p[:, 3:4] + p[:, 5:6]
        energy = p_cont * e_cont + p_uncont * e_uncont
        gate = jax.nn.sigmoid(energy[:, 0:1] - 4.0)
        p_cas = p[:, 0:1]
        p_track = p[:, 1:2] + p[:, 2:3] + p[:, 3:4] + p[:, 5:6]
        dir_pred = (p_cas * d_cas
                    + p_track * (1.0 - gate) * d_low
                    + p_track * gate * d_high)
        out_ref[...] = jnp.concatenate([morph, energy, dir_pred], axis=1)


def _full(shape):
    nd = len(shape)
    return pl.BlockSpec(shape, lambda i: (0,) * nd)


@jax.jit
def _run(coords, features, ids3, w1s, b1s, w2s, b2s):
    nblk = N // BLK
    in_specs = (
        [pl.BlockSpec((BLK, 3), lambda i: (i, 0)),
         pl.BlockSpec((BLK, 128), lambda i: (i, 0)),
         pl.BlockSpec((1, 1, BLK), lambda i: (i, 0, 0))]
        + [_full(w.shape) for w in w1s]
        + [_full(b.shape) for b in b1s]
        + [_full(w.shape) for w in w2s]
        + [_full(b.shape) for b in b2s]
    )
    return pl.pallas_call(
        _fused_kernel,
        grid=(nblk,),
        in_specs=in_specs,
        out_specs=pl.BlockSpec((B, 11), lambda i: (0, 0)),
        out_shape=jax.ShapeDtypeStruct((B, 11), jnp.float32),
        scratch_shapes=[
            pltpu.VMEM((D_IN, D_HALL), jnp.bfloat16),
            pltpu.VMEM((BLK, D_IN), jnp.bfloat16),
            pltpu.VMEM((B, D_HALL), jnp.float32),
            pltpu.VMEM((B, 1), jnp.float32),
        ],
    )(coords, features, ids3, *w1s, *b1s, *w2s, *b2s)


def kernel(coords, features, params, batch_ids):
    order = ("router", "e_contained", "e_uncontained",
             "d_cascade", "d_low", "d_high")
    ids3 = batch_ids.astype(jnp.int32).reshape(N // BLK, 1, BLK)
    w1s = [params[k]["W1"] for k in order]
    b1s = [params[k]["b1"][None, :] for k in order]
    w2s = [params[k]["W2"] for k in order]
    b2s = [params[k]["b2"][None, :] for k in order]
    return _run(coords, features, ids3, w1s, b1s, w2s, b2s)
